# Initial kernel scaffold; baseline (speedup 1.0000x reference)
import jax
import jax.numpy as jnp
from jax.experimental import pallas as pl


def kernel(hidden_states, top_k_index, top_k_weights, gate_up_proj, down_proj):
    raise NotImplementedError("write your pallas kernel here")


# speedup vs baseline: 2.5164x; 2.5164x over previous
import functools

import jax
import jax.numpy as jnp
from jax.experimental import pallas as pl
from jax.experimental.pallas import tpu as pltpu

NUM_EXPERTS = 64
HIDDEN = 768
FF = 512
TOKENS = 2048
BLOCK = 128
GRID = TOKENS // BLOCK + NUM_EXPERTS
DUMP = TOKENS


def _moe_kernel(nblk_ref, bexp_ref, rid_ref,
                w_ref, x_ref, gu_w_ref, dn_w_ref,
                out_ref,
                x_s, y_s):
    b = pl.program_id(0)

    @pl.when(b < nblk_ref[0])
    def _():
        base = b * BLOCK

        def gather_row(r, _):
            t = rid_ref[base + r]
            g = jnp.minimum(t, TOKENS - 1)
            x_s[pl.ds(r, 1), :] = x_ref[pl.ds(g, 1), :]
            return 0

        jax.lax.fori_loop(0, BLOCK, gather_row, 0, unroll=8)

        x = x_s[...]
        gu = jax.lax.dot_general(
            x, gu_w_ref[0], (((1,), (1,)), ((), ())),
            preferred_element_type=jnp.float32)
        gate = gu[:, :FF]
        up = gu[:, FF:]
        h = gate * jax.nn.sigmoid(gate) * up
        y = jax.lax.dot_general(
            h, dn_w_ref[0], (((1,), (1,)), ((), ())),
            preferred_element_type=jnp.float32)
        y_s[...] = y * w_ref[0, 0, :][:, None]

        def scatter_row(r, _):
            t = rid_ref[base + r]
            out_ref[pl.ds(t, 1), :] = y_s[pl.ds(r, 1), :]
            return 0

        jax.lax.fori_loop(0, BLOCK, scatter_row, 0, unroll=8)


@jax.jit
def kernel(hidden_states, top_k_index, top_k_weights, gate_up_proj, down_proj):
    e = top_k_index[:, 0].astype(jnp.int32)
    sort_idx = jnp.argsort(e).astype(jnp.int32)
    counts = jnp.bincount(e, length=NUM_EXPERTS).astype(jnp.int32)
    nb = (counts + BLOCK - 1) // BLOCK
    start_blk = jnp.cumsum(nb) - nb
    nblocks = jnp.sum(nb).astype(jnp.int32)
    offsets = jnp.cumsum(counts) - counts

    b_ids = jnp.arange(GRID, dtype=jnp.int32)
    e_of_b = (jnp.searchsorted(start_blk, b_ids, side="right") - 1).astype(
        jnp.int32)
    p = (b_ids - start_blk[e_of_b])[:, None] * BLOCK + jnp.arange(
        BLOCK, dtype=jnp.int32)[None, :]
    valid = p < counts[e_of_b][:, None]
    spos = jnp.clip(offsets[e_of_b][:, None] + p, 0, TOKENS - 1)
    tok = jnp.where(valid, sort_idx[spos], DUMP).astype(jnp.int32)
    wts = jnp.where(valid, top_k_weights[jnp.clip(tok, 0, TOKENS - 1), 0], 0.0)
    wts = wts.astype(jnp.float32).reshape(GRID, 1, BLOCK)

    grid_spec = pltpu.PrefetchScalarGridSpec(
        num_scalar_prefetch=3,
        grid=(GRID,),
        in_specs=[
            pl.BlockSpec((1, 1, BLOCK), lambda b, n, be, rid: (b, 0, 0)),
            pl.BlockSpec((TOKENS, HIDDEN), lambda b, n, be, rid: (0, 0)),
            pl.BlockSpec((1, 2 * FF, HIDDEN),
                         lambda b, n, be, rid: (be[b], 0, 0)),
            pl.BlockSpec((1, HIDDEN, FF),
                         lambda b, n, be, rid: (be[b], 0, 0)),
        ],
        out_specs=pl.BlockSpec((TOKENS + 8, HIDDEN),
                               lambda b, n, be, rid: (0, 0)),
        scratch_shapes=[
            pltpu.VMEM((BLOCK, HIDDEN), jnp.float32),
            pltpu.VMEM((BLOCK, HIDDEN), jnp.float32),
        ],
    )

    out = pl.pallas_call(
        _moe_kernel,
        grid_spec=grid_spec,
        out_shape=jax.ShapeDtypeStruct((TOKENS + 8, HIDDEN), jnp.float32),
    )(nblocks.reshape(1), e_of_b, tok.reshape(-1),
      wts, hidden_states, gate_up_proj, down_proj)
    return out[:TOKENS]
